# baseline (speedup 1.0000x reference)
import functools

import jax
import jax.numpy as jnp
from jax import lax
from jax.experimental import pallas as pl
from jax.experimental.pallas import tpu as pltpu
from jax.experimental.pallas import tpu_sc as plsc

NUM_SEG = 512
D = 128
B1 = 1024
C = 128
NC, NS = 2, 16
NW = NC * NS
NV = D // 16


def _gate_body(x_ref, w1_ref, b1_ref, w2_ref, e_ref):
    h = jnp.tanh(
        jax.lax.dot_general(x_ref[...], w1_ref[...], (((1,), (0,)), ((), ())),
                            preferred_element_type=jnp.float32)
        + b1_ref[...])
    w2r = w2_ref[...]
    s = jax.lax.dot_general(w2r, h, (((1,), (1,)), ((), ())),
                            preferred_element_type=jnp.float32)
    c = jnp.sum(jnp.abs(w2r))
    e_ref[...] = jnp.exp(s - c)[None]


def _gate(x, W1, b1r, w2r):
    n, d = x.shape
    nb = pl.cdiv(n, B1)
    e2 = pl.pallas_call(
        _gate_body,
        grid=(nb,),
        in_specs=[
            pl.BlockSpec((B1, d), lambda i: (i, 0)),
            pl.BlockSpec((d, d), lambda i: (0, 0)),
            pl.BlockSpec((1, d), lambda i: (0, 0)),
            pl.BlockSpec((1, d), lambda i: (0, 0)),
        ],
        out_specs=pl.BlockSpec((1, 1, B1), lambda i: (i, 0, 0)),
        out_shape=jax.ShapeDtypeStruct((nb, 1, B1), jnp.float32),
    )(x, W1, b1r, w2r)
    return e2.reshape(nb * B1)


def _sc_body(n_rows, pt, kmax, x_hbm, e_hbm, ids_hbm, p_hbm, pd_hbm,
             xbuf, eall, iall, acc, dacc, semx, seme, semi):
    wid = lax.axis_index("s") * NC + lax.axis_index("c")
    base_row = wid * pt
    zv = jnp.zeros((16,), jnp.float32)

    eb0 = jnp.minimum(base_row, n_rows - pt)
    pltpu.make_async_copy(e_hbm.at[pl.ds(eb0, pt)], eall, seme).start()
    pltpu.make_async_copy(ids_hbm.at[pl.ds(eb0, pt)], iall, semi).start()

    def _zero(r, _):
        for v in range(NV):
            acc[r, pl.ds(16 * v, 16)] = zv
        return 0
    lax.fori_loop(0, NUM_SEG, _zero, 0)

    def _zero_d(r, _):
        for v in range(NV):
            dacc[r, pl.ds(16 * v, 16)] = zv
        return 0
    lax.fori_loop(0, NUM_SEG // 8, _zero_d, 0)

    def _dpos(g):
        return jnp.right_shift(g, 3), pl.ds(jnp.bitwise_and(g, 7) * 16, 16)

    def _xcopy(k):
        slot = lax.rem(k, 2)
        s = jnp.minimum(base_row + k * C, n_rows - C)
        return pltpu.make_async_copy(x_hbm.at[pl.ds(s, C)], xbuf.at[slot],
                                     semx.at[slot])

    def _flush_store(cur, accs, dv):
        @pl.when(cur >= 0)
        def _():
            for v in range(NV):
                sl = pl.ds(16 * v, 16)
                acc[cur, sl] = acc[cur, sl] + accs[v]
            dr, dsl = _dpos(cur)
            dacc[dr, dsl] = dacc[dr, dsl] + dv

    kv = jnp.clip((n_rows - base_row + C - 1) // C, 0, kmax)

    @pl.when(kv > 0)
    def _():
        _xcopy(0).start()

    pltpu.make_async_copy(e_hbm.at[pl.ds(eb0, pt)], eall, seme).wait()
    pltpu.make_async_copy(ids_hbm.at[pl.ds(eb0, pt)], iall, semi).wait()

    def _chunk(k, carry):
        slot = lax.rem(k, 2)
        s0 = base_row + k * C
        s = jnp.minimum(s0, n_rows - C)
        o = s - eb0
        _xcopy(k).wait()

        @pl.when(k + 1 < kv)
        def _():
            _xcopy(k + 1).start()

        def _group(g, c):
            cur, accs, dv = c
            b32 = g * 32
            e16s, id16s = [], []
            for h in range(2):
                b16 = b32 + h * 16
                ids16 = iall[pl.ds(o + b16, 16)]
                e16 = eall[pl.ds(o + b16, 16)]
                rowg = s + b16 + lax.iota(jnp.int32, 16)
                e16s.append(jnp.where(rowg >= s0, e16, 0.0))
                id16s.append(ids16)
            first = id16s[0][0]
            last = id16s[1][15]
            uniform = first == last
            need_flush = (first != cur) | jnp.logical_not(uniform)

            @pl.when(need_flush)
            def _():
                _flush_store(cur, accs, dv)

            zero_if_flush = jnp.where(need_flush, 0.0, 1.0)
            av = [a * zero_if_flush for a in accs]
            dv = dv * zero_if_flush

            umask = jnp.where(uniform, 1.0, 0.0)
            for h in range(2):
                e16u = e16s[h] * umask
                dv = dv + e16u
                for r in range(16):
                    er = e16u[r]
                    row = b32 + h * 16 + r
                    for v in range(NV):
                        av[v] = av[v] + er * xbuf[slot, row, pl.ds(16 * v, 16)]

            @pl.when(jnp.logical_not(uniform))
            def _():
                for h in range(2):
                    for r in range(16):
                        idr = id16s[h][r]
                        er = e16s[h][r]
                        row = b32 + h * 16 + r
                        for v in range(NV):
                            sl = pl.ds(16 * v, 16)
                            acc[idr, sl] = acc[idr, sl] + er * xbuf[slot, row, sl]
                        dr, dsl = _dpos(idr)
                        dacc[dr, dsl] = dacc[dr, dsl] + jnp.where(
                            lax.iota(jnp.int32, 16) == 0, er, 0.0)

            return last, tuple(av), dv

        return lax.fori_loop(0, C // 32, _group, carry)

    carry = (jnp.int32(-1), (zv,) * NV, zv)
    carry = lax.fori_loop(0, kv, _chunk, carry)
    _flush_store(*carry)

    pltpu.sync_copy(acc, p_hbm.at[wid])
    pltpu.sync_copy(dacc, pd_hbm.at[wid])


def _sc_pool(x, e1, batch):
    n, d = x.shape
    pt = ((n + NW * C - 1) // (NW * C)) * C
    kmax = pt // C
    mesh = plsc.VectorSubcoreMesh(core_axis_name="c", subcore_axis_name="s",
                                  num_cores=NC, num_subcores=NS)
    f = pl.kernel(
        functools.partial(_sc_body, n, pt, kmax),
        out_type=(jax.ShapeDtypeStruct((NW, NUM_SEG, D), jnp.float32),
                  jax.ShapeDtypeStruct((NW, NUM_SEG // 8, D), jnp.float32)),
        mesh=mesh,
        scratch_types=[
            pltpu.VMEM((2, C, D), jnp.float32),
            pltpu.VMEM((pt,), jnp.float32),
            pltpu.VMEM((pt,), jnp.int32),
            pltpu.VMEM((NUM_SEG, D), jnp.float32),
            pltpu.VMEM((NUM_SEG // 8, D), jnp.float32),
            pltpu.SemaphoreType.DMA((2,)),
            pltpu.SemaphoreType.DMA,
            pltpu.SemaphoreType.DMA,
        ],
    )
    return f(x, e1, batch)


def _epi_body(p_ref, pd_ref, o_ref):
    psum = jnp.sum(p_ref[...], axis=0)
    den = jnp.sum(jnp.sum(pd_ref[...], axis=0), axis=1, keepdims=True)
    o_ref[...] = jnp.where(den > 0, psum / den, 0.0)


def _epilogue(p, pd):
    return pl.pallas_call(
        _epi_body,
        grid=(1,),
        in_specs=[
            pl.BlockSpec((NW, NUM_SEG, D), lambda i: (0, 0, 0)),
            pl.BlockSpec((NW, NUM_SEG, 16), lambda i: (0, 0, 0)),
        ],
        out_specs=pl.BlockSpec((NUM_SEG, D), lambda i: (0, 0)),
        out_shape=jax.ShapeDtypeStruct((NUM_SEG, D), jnp.float32),
    )(p, pd)


def kernel(x, batch, W1, b1, W2, b2):
    del b2
    n, d = x.shape
    e1 = _gate(x, W1, b1.reshape(1, d), W2.reshape(1, d))
    p, pd = _sc_pool(x, e1, batch)
    return _epilogue(p, pd.reshape(NW, NUM_SEG, 16))


# speedup vs baseline: 1.0895x; 1.0895x over previous
import functools

import jax
import jax.numpy as jnp
from jax import lax
from jax.experimental import pallas as pl
from jax.experimental.pallas import tpu as pltpu
from jax.experimental.pallas import tpu_sc as plsc

NUM_SEG = 512
D = 128
B1 = 1024
C = 128
NC, NS = 2, 16
NW = NC * NS
NV = D // 16


def _gate_body(x_ref, w1_ref, b1_ref, w2_ref, e_ref):
    h = jnp.tanh(
        jax.lax.dot_general(x_ref[...], w1_ref[...], (((1,), (0,)), ((), ())),
                            preferred_element_type=jnp.float32)
        + b1_ref[...])
    w2r = w2_ref[...]
    s = jax.lax.dot_general(w2r, h, (((1,), (1,)), ((), ())),
                            preferred_element_type=jnp.float32)
    c = jnp.sum(jnp.abs(w2r))
    e_ref[...] = jnp.exp(s - c)[None]


def _gate(x, W1, b1r, w2r):
    n, d = x.shape
    nb = pl.cdiv(n, B1)
    e2 = pl.pallas_call(
        _gate_body,
        grid=(nb,),
        in_specs=[
            pl.BlockSpec((B1, d), lambda i: (i, 0)),
            pl.BlockSpec((d, d), lambda i: (0, 0)),
            pl.BlockSpec((1, d), lambda i: (0, 0)),
            pl.BlockSpec((1, d), lambda i: (0, 0)),
        ],
        out_specs=pl.BlockSpec((1, 1, B1), lambda i: (i, 0, 0)),
        out_shape=jax.ShapeDtypeStruct((nb, 1, B1), jnp.float32),
    )(x, W1, b1r, w2r)
    return e2.reshape(nb * B1)


def _sc_body(n_rows, pt, kmax, x_hbm, e_hbm, ids_hbm, p_hbm, pd_hbm,
             xbuf, eall, iall, acc, dacc, semx, seme, semi):
    wid = lax.axis_index("s") * NC + lax.axis_index("c")
    base_row = wid * pt
    zv = jnp.zeros((16,), jnp.float32)

    eb0 = jnp.minimum(base_row, n_rows - pt)
    pltpu.make_async_copy(e_hbm.at[pl.ds(eb0, pt)], eall, seme).start()
    pltpu.make_async_copy(ids_hbm.at[pl.ds(eb0, pt)], iall, semi).start()

    def _zero(r, _):
        for v in range(NV):
            acc[r, pl.ds(16 * v, 16)] = zv
        return 0
    lax.fori_loop(0, NUM_SEG, _zero, 0)

    def _zero_d(r, _):
        for v in range(NV):
            dacc[r, pl.ds(16 * v, 16)] = zv
        return 0
    lax.fori_loop(0, NUM_SEG // 8, _zero_d, 0)

    def _dpos(g):
        return jnp.right_shift(g, 3), pl.ds(jnp.bitwise_and(g, 7) * 16, 16)

    def _xcopy(k):
        slot = lax.rem(k, 2)
        s = jnp.minimum(base_row + k * C, n_rows - C)
        return pltpu.make_async_copy(x_hbm.at[pl.ds(s, C)], xbuf.at[slot],
                                     semx.at[slot])

    def _flush_store(cur, accs, dv):
        @pl.when(cur >= 0)
        def _():
            for v in range(NV):
                sl = pl.ds(16 * v, 16)
                acc[cur, sl] = acc[cur, sl] + accs[v]
            dr, dsl = _dpos(cur)
            dacc[dr, dsl] = dacc[dr, dsl] + dv

    kv = jnp.clip((n_rows - base_row + C - 1) // C, 0, kmax)

    @pl.when(kv > 0)
    def _():
        _xcopy(0).start()

    pltpu.make_async_copy(e_hbm.at[pl.ds(eb0, pt)], eall, seme).wait()
    pltpu.make_async_copy(ids_hbm.at[pl.ds(eb0, pt)], iall, semi).wait()

    def _chunk(k, carry):
        slot = lax.rem(k, 2)
        s0 = base_row + k * C
        s = jnp.minimum(s0, n_rows - C)
        o = s - eb0
        _xcopy(k).wait()

        @pl.when(k + 1 < kv)
        def _():
            _xcopy(k + 1).start()

        def _group(g, c):
            cur, accs, dv = c
            b16 = g * 16
            ids16 = iall[pl.ds(o + b16, 16)]
            e16 = eall[pl.ds(o + b16, 16)]
            rowg = s + b16 + lax.iota(jnp.int32, 16)
            e16 = jnp.where(rowg >= s0, e16, 0.0)
            first = ids16[0]
            last = ids16[15]
            uniform = first == last
            need_flush = (first != cur) | jnp.logical_not(uniform)

            @pl.when(need_flush)
            def _():
                _flush_store(cur, accs, dv)

            zero_if_flush = jnp.where(need_flush, 0.0, 1.0)
            av = [a * zero_if_flush for a in accs]
            dv = dv * zero_if_flush

            e16u = e16 * jnp.where(uniform, 1.0, 0.0)
            dv = dv + e16u
            for r in range(16):
                er = e16u[r]
                for v in range(NV):
                    av[v] = av[v] + er * xbuf[slot, b16 + r, pl.ds(16 * v, 16)]

            @pl.when(jnp.logical_not(uniform))
            def _():
                for r in range(16):
                    idr = ids16[r]
                    er = e16[r]
                    for v in range(NV):
                        sl = pl.ds(16 * v, 16)
                        acc[idr, sl] = acc[idr, sl] + er * xbuf[slot, b16 + r, sl]
                    dr, dsl = _dpos(idr)
                    dacc[dr, dsl] = dacc[dr, dsl] + jnp.where(
                        lax.iota(jnp.int32, 16) == 0, er, 0.0)

            return last, tuple(av), dv

        return lax.fori_loop(0, C // 16, _group, carry)

    carry = (jnp.int32(-1), (zv,) * NV, zv)
    carry = lax.fori_loop(0, kv, _chunk, carry)
    _flush_store(*carry)

    pltpu.sync_copy(acc, p_hbm.at[wid])
    pltpu.sync_copy(dacc, pd_hbm.at[wid])


def _sc_pool(x, e1, batch):
    n, d = x.shape
    pt = ((n + NW * C - 1) // (NW * C)) * C
    kmax = pt // C
    mesh = plsc.VectorSubcoreMesh(core_axis_name="c", subcore_axis_name="s",
                                  num_cores=NC, num_subcores=NS)
    f = pl.kernel(
        functools.partial(_sc_body, n, pt, kmax),
        out_type=(jax.ShapeDtypeStruct((NW, NUM_SEG, D), jnp.float32),
                  jax.ShapeDtypeStruct((NW, NUM_SEG // 8, D), jnp.float32)),
        mesh=mesh,
        scratch_types=[
            pltpu.VMEM((2, C, D), jnp.float32),
            pltpu.VMEM((pt,), jnp.float32),
            pltpu.VMEM((pt,), jnp.int32),
            pltpu.VMEM((NUM_SEG, D), jnp.float32),
            pltpu.VMEM((NUM_SEG // 8, D), jnp.float32),
            pltpu.SemaphoreType.DMA((2,)),
            pltpu.SemaphoreType.DMA,
            pltpu.SemaphoreType.DMA,
        ],
    )
    return f(x, e1, batch)


def _epi_body(p_ref, pd_ref, o_ref):
    psum = jnp.sum(p_ref[...], axis=0)
    den = jnp.sum(jnp.sum(pd_ref[...], axis=0), axis=1, keepdims=True)
    o_ref[...] = jnp.where(den > 0, psum / den, 0.0)


def _epilogue(p, pd):
    return pl.pallas_call(
        _epi_body,
        grid=(1,),
        in_specs=[
            pl.BlockSpec((NW, NUM_SEG, D), lambda i: (0, 0, 0)),
            pl.BlockSpec((NW, NUM_SEG, 16), lambda i: (0, 0, 0)),
        ],
        out_specs=pl.BlockSpec((NUM_SEG, D), lambda i: (0, 0)),
        out_shape=jax.ShapeDtypeStruct((NUM_SEG, D), jnp.float32),
    )(p, pd)


def kernel(x, batch, W1, b1, W2, b2):
    del b2
    n, d = x.shape
    e1 = _gate(x, W1, b1.reshape(1, d), W2.reshape(1, d))
    p, pd = _sc_pool(x, e1, batch)
    return _epilogue(p, pd.reshape(NW, NUM_SEG, 16))


# speedup vs baseline: 1.2795x; 1.1744x over previous
import functools

import jax
import jax.numpy as jnp
from jax import lax
from jax.experimental import pallas as pl
from jax.experimental.pallas import tpu as pltpu
from jax.experimental.pallas import tpu_sc as plsc

NUM_SEG = 512
D = 128
B1 = 2048
C = 128
NC, NS = 2, 16
NW = NC * NS
NV = D // 16


def _gate_body(x_ref, w1_ref, b1_ref, w2_ref, e_ref):
    h = jnp.tanh(
        jax.lax.dot_general(x_ref[...], w1_ref[...], (((1,), (0,)), ((), ())),
                            preferred_element_type=jnp.float32)
        + b1_ref[...])
    w2r = w2_ref[...]
    s = jax.lax.dot_general(w2r, h, (((1,), (1,)), ((), ())),
                            preferred_element_type=jnp.float32)
    c = jnp.sum(jnp.abs(w2r))
    e_ref[...] = jnp.exp(s - c)[None]


def _gate(x, W1, b1r, w2r):
    n, d = x.shape
    nb = pl.cdiv(n, B1)
    e2 = pl.pallas_call(
        _gate_body,
        grid=(nb,),
        in_specs=[
            pl.BlockSpec((B1, d), lambda i: (i, 0)),
            pl.BlockSpec((d, d), lambda i: (0, 0)),
            pl.BlockSpec((1, d), lambda i: (0, 0)),
            pl.BlockSpec((1, d), lambda i: (0, 0)),
        ],
        out_specs=pl.BlockSpec((1, 1, B1), lambda i: (i, 0, 0)),
        out_shape=jax.ShapeDtypeStruct((nb, 1, B1), jnp.float32),
    )(x, W1, b1r, w2r)
    return e2.reshape(nb * B1)


def _sc_body(n_rows, pt, kmax, x_hbm, e_hbm, ids_hbm, p_hbm, pd_hbm,
             xbuf, eall, iall, acc, dacc, semx, seme, semi):
    wid = lax.axis_index("s") * NC + lax.axis_index("c")
    base_row = wid * pt
    zv = jnp.zeros((16,), jnp.float32)

    eb0 = jnp.minimum(base_row, n_rows - pt)
    pltpu.make_async_copy(e_hbm.at[pl.ds(eb0, pt)], eall, seme).start()
    pltpu.make_async_copy(ids_hbm.at[pl.ds(eb0, pt)], iall, semi).start()

    def _zero(r, _):
        for v in range(NV):
            acc[r, pl.ds(16 * v, 16)] = zv
        return 0
    lax.fori_loop(0, NUM_SEG, _zero, 0)

    def _zero_d(r, _):
        for v in range(NV):
            dacc[r, pl.ds(16 * v, 16)] = zv
        return 0
    lax.fori_loop(0, NUM_SEG // 8, _zero_d, 0)

    def _dpos(g):
        return jnp.right_shift(g, 3), pl.ds(jnp.bitwise_and(g, 7) * 16, 16)

    def _xcopy(k):
        slot = lax.rem(k, 2)
        s = jnp.minimum(base_row + k * C, n_rows - C)
        return pltpu.make_async_copy(x_hbm.at[pl.ds(s, C)], xbuf.at[slot],
                                     semx.at[slot])

    def _flush_store(cur, accs, dv):
        @pl.when(cur >= 0)
        def _():
            for v in range(NV):
                sl = pl.ds(16 * v, 16)
                acc[cur, sl] = acc[cur, sl] + accs[v]
            dr, dsl = _dpos(cur)
            dacc[dr, dsl] = dacc[dr, dsl] + dv

    kv = jnp.clip((n_rows - base_row + C - 1) // C, 0, kmax)

    @pl.when(kv > 0)
    def _():
        _xcopy(0).start()

    pltpu.make_async_copy(e_hbm.at[pl.ds(eb0, pt)], eall, seme).wait()
    pltpu.make_async_copy(ids_hbm.at[pl.ds(eb0, pt)], iall, semi).wait()

    def _chunk(k, carry):
        slot = lax.rem(k, 2)
        s0 = base_row + k * C
        s = jnp.minimum(s0, n_rows - C)
        o = s - eb0
        _xcopy(k).wait()

        @pl.when(k + 1 < kv)
        def _():
            _xcopy(k + 1).start()

        def _group(g, c):
            cur, accs, dv = c
            b16 = g * 16
            ids16 = iall[pl.ds(o + b16, 16)]
            e16 = eall[pl.ds(o + b16, 16)]
            rowg = s + b16 + lax.iota(jnp.int32, 16)
            e16 = jnp.where(rowg >= s0, e16, 0.0)
            first = ids16[0]
            last = ids16[15]
            uniform = first == last
            need_flush = (first != cur) | jnp.logical_not(uniform)

            @pl.when(need_flush)
            def _():
                _flush_store(cur, accs, dv)

            zero_if_flush = jnp.where(need_flush, 0.0, 1.0)
            av = [a * zero_if_flush for a in accs]
            dv = dv * zero_if_flush

            e16u = e16 * jnp.where(uniform, 1.0, 0.0)
            dv = dv + e16u
            for r in range(16):
                er = e16u[r]
                for v in range(NV):
                    av[v] = av[v] + er * xbuf[slot, b16 + r, pl.ds(16 * v, 16)]

            @pl.when(jnp.logical_not(uniform))
            def _():
                for r in range(16):
                    idr = ids16[r]
                    er = e16[r]
                    for v in range(NV):
                        sl = pl.ds(16 * v, 16)
                        acc[idr, sl] = acc[idr, sl] + er * xbuf[slot, b16 + r, sl]
                    dr, dsl = _dpos(idr)
                    dacc[dr, dsl] = dacc[dr, dsl] + jnp.where(
                        lax.iota(jnp.int32, 16) == 0, er, 0.0)

            return last, tuple(av), dv

        return lax.fori_loop(0, C // 16, _group, carry)

    carry = (jnp.int32(-1), (zv,) * NV, zv)
    carry = lax.fori_loop(0, kv, _chunk, carry)
    _flush_store(*carry)

    pltpu.sync_copy(acc, p_hbm.at[wid])
    pltpu.sync_copy(dacc, pd_hbm.at[wid])


def _sc_pool(x, e1, batch):
    n, d = x.shape
    pt = ((n + NW * C - 1) // (NW * C)) * C
    kmax = pt // C
    mesh = plsc.VectorSubcoreMesh(core_axis_name="c", subcore_axis_name="s",
                                  num_cores=NC, num_subcores=NS)
    f = pl.kernel(
        functools.partial(_sc_body, n, pt, kmax),
        out_type=(jax.ShapeDtypeStruct((NW, NUM_SEG, D), jnp.float32),
                  jax.ShapeDtypeStruct((NW, NUM_SEG // 8, D), jnp.float32)),
        mesh=mesh,
        scratch_types=[
            pltpu.VMEM((2, C, D), jnp.float32),
            pltpu.VMEM((pt,), jnp.float32),
            pltpu.VMEM((pt,), jnp.int32),
            pltpu.VMEM((NUM_SEG, D), jnp.float32),
            pltpu.VMEM((NUM_SEG // 8, D), jnp.float32),
            pltpu.SemaphoreType.DMA((2,)),
            pltpu.SemaphoreType.DMA,
            pltpu.SemaphoreType.DMA,
        ],
    )
    return f(x, e1, batch)


def _epi_body(p_ref, pd_ref, o_ref):
    psum = jnp.sum(p_ref[...], axis=0)
    den = jnp.sum(jnp.sum(pd_ref[...], axis=0), axis=1, keepdims=True)
    o_ref[...] = jnp.where(den > 0, psum / den, 0.0)


def _epilogue(p, pd):
    return pl.pallas_call(
        _epi_body,
        grid=(1,),
        in_specs=[
            pl.BlockSpec((NW, NUM_SEG, D), lambda i: (0, 0, 0)),
            pl.BlockSpec((NW, NUM_SEG, 16), lambda i: (0, 0, 0)),
        ],
        out_specs=pl.BlockSpec((NUM_SEG, D), lambda i: (0, 0)),
        out_shape=jax.ShapeDtypeStruct((NUM_SEG, D), jnp.float32),
    )(p, pd)


def kernel(x, batch, W1, b1, W2, b2):
    del b2
    n, d = x.shape
    e1 = _gate(x, W1, b1.reshape(1, d), W2.reshape(1, d))
    p, pd = _sc_pool(x, e1, batch)
    return _epilogue(p, pd.reshape(NW, NUM_SEG, 16))


# speedup vs baseline: 1.3998x; 1.0940x over previous
import functools

import jax
import jax.numpy as jnp
from jax import lax
from jax.experimental import pallas as pl
from jax.experimental.pallas import tpu as pltpu
from jax.experimental.pallas import tpu_sc as plsc

NUM_SEG = 512
D = 128
B1 = 4096
C = 128
NC, NS = 2, 16
NW = NC * NS
NV = D // 16


def _gate_body(x_ref, w1_ref, b1_ref, w2_ref, e_ref):
    h = jnp.tanh(
        jax.lax.dot_general(x_ref[...], w1_ref[...], (((1,), (0,)), ((), ())),
                            preferred_element_type=jnp.float32)
        + b1_ref[...])
    w2r = w2_ref[...]
    s = jax.lax.dot_general(w2r, h, (((1,), (1,)), ((), ())),
                            preferred_element_type=jnp.float32)
    c = jnp.sum(jnp.abs(w2r))
    e_ref[...] = jnp.exp(s - c)[None]


def _gate(x, W1, b1r, w2r):
    n, d = x.shape
    nb = pl.cdiv(n, B1)
    e2 = pl.pallas_call(
        _gate_body,
        grid=(nb,),
        in_specs=[
            pl.BlockSpec((B1, d), lambda i: (i, 0)),
            pl.BlockSpec((d, d), lambda i: (0, 0)),
            pl.BlockSpec((1, d), lambda i: (0, 0)),
            pl.BlockSpec((1, d), lambda i: (0, 0)),
        ],
        out_specs=pl.BlockSpec((1, 1, B1), lambda i: (i, 0, 0)),
        out_shape=jax.ShapeDtypeStruct((nb, 1, B1), jnp.float32),
    )(x, W1, b1r, w2r)
    return e2.reshape(nb * B1)


def _sc_body(n_rows, pt, kmax, x_hbm, e_hbm, ids_hbm, p_hbm, pd_hbm,
             xbuf, eall, iall, acc, dacc, semx, seme, semi):
    wid = lax.axis_index("s") * NC + lax.axis_index("c")
    base_row = wid * pt
    zv = jnp.zeros((16,), jnp.float32)

    eb0 = jnp.minimum(base_row, n_rows - pt)
    pltpu.make_async_copy(e_hbm.at[pl.ds(eb0, pt)], eall, seme).start()
    pltpu.make_async_copy(ids_hbm.at[pl.ds(eb0, pt)], iall, semi).start()

    def _zero(r, _):
        for v in range(NV):
            acc[r, pl.ds(16 * v, 16)] = zv
        return 0
    lax.fori_loop(0, NUM_SEG, _zero, 0)

    def _zero_d(r, _):
        for v in range(NV):
            dacc[r, pl.ds(16 * v, 16)] = zv
        return 0
    lax.fori_loop(0, NUM_SEG // 8, _zero_d, 0)

    def _dpos(g):
        return jnp.right_shift(g, 3), pl.ds(jnp.bitwise_and(g, 7) * 16, 16)

    def _xcopy(k):
        slot = lax.rem(k, 2)
        s = jnp.minimum(base_row + k * C, n_rows - C)
        return pltpu.make_async_copy(x_hbm.at[pl.ds(s, C)], xbuf.at[slot],
                                     semx.at[slot])

    def _flush_store(cur, accs, dv):
        @pl.when(cur >= 0)
        def _():
            for v in range(NV):
                sl = pl.ds(16 * v, 16)
                acc[cur, sl] = acc[cur, sl] + accs[v]
            dr, dsl = _dpos(cur)
            dacc[dr, dsl] = dacc[dr, dsl] + dv

    kv = jnp.clip((n_rows - base_row + C - 1) // C, 0, kmax)

    @pl.when(kv > 0)
    def _():
        _xcopy(0).start()

    pltpu.make_async_copy(e_hbm.at[pl.ds(eb0, pt)], eall, seme).wait()
    pltpu.make_async_copy(ids_hbm.at[pl.ds(eb0, pt)], iall, semi).wait()

    def _chunk(k, carry):
        slot = lax.rem(k, 2)
        s0 = base_row + k * C
        s = jnp.minimum(s0, n_rows - C)
        o = s - eb0
        _xcopy(k).wait()

        @pl.when(k + 1 < kv)
        def _():
            _xcopy(k + 1).start()

        def _group(g, c):
            cur, accs, dv = c
            b16 = g * 16
            ids16 = iall[pl.ds(o + b16, 16)]
            e16 = eall[pl.ds(o + b16, 16)]
            rowg = s + b16 + lax.iota(jnp.int32, 16)
            e16 = jnp.where(rowg >= s0, e16, 0.0)
            first = ids16[0]
            last = ids16[15]
            uniform = first == last
            need_flush = (first != cur) | jnp.logical_not(uniform)

            @pl.when(need_flush)
            def _():
                _flush_store(cur, accs, dv)

            zero_if_flush = jnp.where(need_flush, 0.0, 1.0)
            av = [a * zero_if_flush for a in accs]
            dv = dv * zero_if_flush

            e16u = e16 * jnp.where(uniform, 1.0, 0.0)
            dv = dv + e16u
            for r in range(16):
                er = e16u[r]
                for v in range(NV):
                    av[v] = av[v] + er * xbuf[slot, b16 + r, pl.ds(16 * v, 16)]

            @pl.when(jnp.logical_not(uniform))
            def _():
                for r in range(16):
                    idr = ids16[r]
                    er = e16[r]
                    for v in range(NV):
                        sl = pl.ds(16 * v, 16)
                        acc[idr, sl] = acc[idr, sl] + er * xbuf[slot, b16 + r, sl]
                    dr, dsl = _dpos(idr)
                    dacc[dr, dsl] = dacc[dr, dsl] + jnp.where(
                        lax.iota(jnp.int32, 16) == 0, er, 0.0)

            return last, tuple(av), dv

        return lax.fori_loop(0, C // 16, _group, carry)

    carry = (jnp.int32(-1), (zv,) * NV, zv)
    carry = lax.fori_loop(0, kv, _chunk, carry)
    _flush_store(*carry)

    pltpu.sync_copy(acc, p_hbm.at[wid])
    pltpu.sync_copy(dacc, pd_hbm.at[wid])


def _sc_pool(x, e1, batch):
    n, d = x.shape
    pt = ((n + NW * C - 1) // (NW * C)) * C
    kmax = pt // C
    mesh = plsc.VectorSubcoreMesh(core_axis_name="c", subcore_axis_name="s",
                                  num_cores=NC, num_subcores=NS)
    f = pl.kernel(
        functools.partial(_sc_body, n, pt, kmax),
        out_type=(jax.ShapeDtypeStruct((NW, NUM_SEG, D), jnp.float32),
                  jax.ShapeDtypeStruct((NW, NUM_SEG // 8, D), jnp.float32)),
        mesh=mesh,
        scratch_types=[
            pltpu.VMEM((2, C, D), jnp.float32),
            pltpu.VMEM((pt,), jnp.float32),
            pltpu.VMEM((pt,), jnp.int32),
            pltpu.VMEM((NUM_SEG, D), jnp.float32),
            pltpu.VMEM((NUM_SEG // 8, D), jnp.float32),
            pltpu.SemaphoreType.DMA((2,)),
            pltpu.SemaphoreType.DMA,
            pltpu.SemaphoreType.DMA,
        ],
    )
    return f(x, e1, batch)


def _epi_body(p_ref, pd_ref, o_ref):
    psum = jnp.sum(p_ref[...], axis=0)
    den = jnp.sum(jnp.sum(pd_ref[...], axis=0), axis=1, keepdims=True)
    o_ref[...] = jnp.where(den > 0, psum / den, 0.0)


def _epilogue(p, pd):
    return pl.pallas_call(
        _epi_body,
        grid=(1,),
        in_specs=[
            pl.BlockSpec((NW, NUM_SEG, D), lambda i: (0, 0, 0)),
            pl.BlockSpec((NW, NUM_SEG, 16), lambda i: (0, 0, 0)),
        ],
        out_specs=pl.BlockSpec((NUM_SEG, D), lambda i: (0, 0)),
        out_shape=jax.ShapeDtypeStruct((NUM_SEG, D), jnp.float32),
    )(p, pd)


def kernel(x, batch, W1, b1, W2, b2):
    del b2
    n, d = x.shape
    e1 = _gate(x, W1, b1.reshape(1, d), W2.reshape(1, d))
    p, pd = _sc_pool(x, e1, batch)
    return _epilogue(p, pd.reshape(NW, NUM_SEG, 16))
